# baseline (speedup 1.0000x reference)
import functools

import jax
import jax.numpy as jnp
from jax import lax
from jax.experimental import pallas as pl
from jax.experimental.pallas import tpu as pltpu
from jax.experimental.pallas import tpu_sc as plsc

N = 10000
E = 320000
F_IN = 128
C = 16
H1 = 8
F1 = 8

NPAD = 10240
NTILES = 32
EPAD = 327680
EDGES_PER_TILE = EPAD // NTILES
CHUNK = 64
ROWS_PER_TILE = NPAD // 16
BLK = 512
GRID = NPAD // BLK
NEG = -1e30


def _tc_pre_body(x_ref, w_ref, a_ref, t_ref, *, f_out, h_heads):
    h = jnp.dot(x_ref[...], w_ref[...], preferred_element_type=jnp.float32)
    al = jnp.dot(h, a_ref[...], preferred_element_type=jnp.float32)
    n = h.shape[0]
    z = jnp.zeros((n, 16 - h_heads), jnp.float32)
    neg = jnp.full((n, 16 - h_heads), NEG, jnp.float32)
    ztail = jnp.zeros((n, 128 - f_out - 32), jnp.float32)
    t_ref[...] = jnp.concatenate(
        [h, al[:, :h_heads], z, al[:, h_heads:], neg, ztail], axis=1)


def _tc_pre(x, w, a, f_out, h_heads):
    f_in = x.shape[1]
    return pl.pallas_call(
        functools.partial(_tc_pre_body, f_out=f_out, h_heads=h_heads),
        grid=(GRID,),
        in_specs=[
            pl.BlockSpec((BLK, f_in), lambda i: (i, 0)),
            pl.BlockSpec((f_in, f_out), lambda i: (0, 0)),
            pl.BlockSpec((f_out, 2 * h_heads), lambda i: (0, 0)),
        ],
        out_specs=pl.BlockSpec((BLK, 128), lambda i: (i, 0)),
        out_shape=jax.ShapeDtypeStruct((NPAD, 128), jnp.float32),
    )(x, w, a)


def _tc_mid_body(a0_ref, a1_ref, p_ref, w2_ref, a2_ref, t_ref):
    a0 = a0_ref[...]
    a1 = a1_ref[...]
    num = a0[:, :64] + a1[:, :64]
    den = a0[:, 64:72] + a1[:, 64:72]
    den_e = jnp.dot(den, p_ref[...], preferred_element_type=jnp.float32)
    out1 = num / (den_e + 1e-16)
    g = jnp.where(out1 > 0, out1, jnp.exp(out1) - 1.0)
    h2 = jnp.dot(g, w2_ref[...], preferred_element_type=jnp.float32)
    al2 = jnp.dot(h2, a2_ref[...], preferred_element_type=jnp.float32)
    n = h2.shape[0]
    z = jnp.zeros((n, 15), jnp.float32)
    neg = jnp.full((n, 15), NEG, jnp.float32)
    ztail = jnp.zeros((n, 80), jnp.float32)
    t_ref[...] = jnp.concatenate(
        [h2, al2[:, 0:1], z, al2[:, 1:2], neg, ztail], axis=1)


def _tc_mid(acc0, acc1, p, w2, a2):
    return pl.pallas_call(
        _tc_mid_body,
        grid=(GRID,),
        in_specs=[
            pl.BlockSpec((BLK, 80), lambda i: (i, 0)),
            pl.BlockSpec((BLK, 80), lambda i: (i, 0)),
            pl.BlockSpec((8, 64), lambda i: (0, 0)),
            pl.BlockSpec((64, 16), lambda i: (0, 0)),
            pl.BlockSpec((16, 2), lambda i: (0, 0)),
        ],
        out_specs=pl.BlockSpec((BLK, 128), lambda i: (i, 0)),
        out_shape=jax.ShapeDtypeStruct((NPAD, 128), jnp.float32),
    )(acc0, acc1, p, w2, a2)


def _tc_post_body(a0_ref, a1_ref, out_ref):
    a0 = a0_ref[...]
    a1 = a1_ref[...]
    num = a0[:, :16] + a1[:, :16]
    den = a0[:, 16:17] + a1[:, 16:17]
    zv = num / (den + 1e-16)
    zm = jnp.max(zv, axis=1, keepdims=True)
    ez = jnp.exp(zv - zm)
    out_ref[...] = ez / jnp.sum(ez, axis=1, keepdims=True)


def _tc_post(acc0, acc1):
    return pl.pallas_call(
        _tc_post_body,
        grid=(GRID,),
        in_specs=[
            pl.BlockSpec((BLK, 32), lambda i: (i, 0)),
            pl.BlockSpec((BLK, 32), lambda i: (i, 0)),
        ],
        out_specs=pl.BlockSpec((BLK, 16), lambda i: (i, 0)),
        out_shape=jax.ShapeDtypeStruct((NPAD, 16), jnp.float32),
    )(acc0, acc1)


def _sc_edge_pass(t, src2d, dst2d, f_out, h_heads):
    w_row = f_out + 16
    nreg = f_out // 16
    shift = {8: 3, 16: 4}[f_out // h_heads]
    chunks_per_tile = EDGES_PER_TILE // CHUNK

    mesh = plsc.VectorSubcoreMesh(
        core_axis_name="c", subcore_axis_name="s", num_cores=2, num_subcores=16)

    @functools.partial(
        pl.kernel,
        out_type=jax.ShapeDtypeStruct((2 * NPAD, w_row), jnp.float32),
        mesh=mesh,
        compiler_params=pltpu.CompilerParams(use_tc_tiling_on_sc=False),
        scratch_types=[
            pltpu.VMEM((chunks_per_tile, CHUNK), jnp.int32),
            pltpu.VMEM((chunks_per_tile, CHUNK), jnp.int32),
            pltpu.VMEM((CHUNK, 128), jnp.float32),
            pltpu.VMEM((CHUNK, 128), jnp.float32),
            pltpu.VMEM((CHUNK, 128), jnp.float32),
            pltpu.VMEM((CHUNK, 128), jnp.float32),
            pltpu.VMEM((CHUNK, w_row), jnp.float32),
            pltpu.VMEM((CHUNK, w_row), jnp.float32),
            pltpu.VMEM((CHUNK,), jnp.int32),
            pltpu.VMEM((CHUNK,), jnp.int32),
            pltpu.VMEM((CHUNK,), jnp.int32),
            pltpu.VMEM((CHUNK,), jnp.int32),
            pltpu.VMEM((CHUNK,), jnp.int32),
            pltpu.VMEM((CHUNK,), jnp.int32),
            pltpu.VMEM_SHARED((NPAD, w_row), jnp.float32),
            pltpu.SemaphoreType.DMA,
            pltpu.SemaphoreType.DMA,
            pltpu.SemaphoreType.DMA,
            pltpu.SemaphoreType.DMA,
            pltpu.SemaphoreType.DMA,
            pltpu.SemaphoreType.DMA,
        ],
    )
    def edge_kernel(t_hbm, src_hbm, dst_hbm, out_hbm,
                    sidx, didx, abuf0, abuf1, bbuf0, bbuf1, msg0, msg1,
                    sidxc0, sidxc1, didxc0, didxc1, didxs0, didxs1,
                    acc, semA0, semA1, semB0, semB1, semS0, semS1):
        abuf = (abuf0, abuf1)
        bbuf = (bbuf0, bbuf1)
        msg_buf = (msg0, msg1)
        sidx_c = (sidxc0, sidxc1)
        didx_c = (didxc0, didxc1)
        didx_s = (didxs0, didxs1)
        semA = (semA0, semA1)
        semB = (semB0, semB1)
        semS = (semS0, semS1)
        cid = lax.axis_index("c")
        sid = lax.axis_index("s")
        wid = cid * 16 + sid

        def zrow(r, _):
            for j in range(w_row // 16):
                msg_buf[0][r, pl.ds(16 * j, 16)] = jnp.zeros((16,),
                                                             jnp.float32)
            return 0
        lax.fori_loop(0, CHUNK, zrow, 0)
        row0 = sid * ROWS_PER_TILE
        for k in range(ROWS_PER_TILE // CHUNK):
            pltpu.sync_copy(msg_buf[0],
                            acc.at[pl.ds(row0 + k * CHUNK, CHUNK)])

        pltpu.sync_copy(src_hbm.at[pl.ds(wid * chunks_per_tile,
                                         chunks_per_tile)], sidx)
        pltpu.sync_copy(dst_hbm.at[pl.ds(wid * chunks_per_tile,
                                         chunks_per_tile)], didx)
        plsc.subcore_barrier()

        def stage_idx(ci, par):
            for q in range(CHUNK // 16):
                sidx_c[par][pl.ds(16 * q, 16)] = sidx[ci, pl.ds(16 * q, 16)]
                didx_c[par][pl.ds(16 * q, 16)] = didx[ci, pl.ds(16 * q, 16)]

        def fire(par):
            pltpu.async_copy(t_hbm.at[sidx_c[par]], abuf[par], semA[par])
            pltpu.async_copy(t_hbm.at[didx_c[par]], bbuf[par], semB[par])

        def gwait(par):
            pltpu.make_async_copy(
                t_hbm.at[sidx_c[par]], abuf[par], semA[par]).wait()
            pltpu.make_async_copy(
                t_hbm.at[didx_c[par]], bbuf[par], semB[par]).wait()

        def compute(par):
            @plsc.parallel_loop(0, CHUNK, 1, unroll=8)
            def edge(b):
                asv = abuf[par][b, pl.ds(f_out, 16)]
                adv = bbuf[par][b, pl.ds(f_out + 16, 16)]
                e = asv + adv
                e = jnp.maximum(e, 0.2 * e)
                w = jnp.exp(e)
                msg_buf[par][b, pl.ds(f_out, 16)] = w
                lane = lax.iota(jnp.int32, 16)
                for j in range(nreg):
                    rep = lax.shift_right_logical(lane + 16 * j, shift)
                    wj = jnp.take_along_axis(w, rep, axis=0)
                    msg_buf[par][b, pl.ds(16 * j, 16)] = (
                        wj * abuf[par][b, pl.ds(16 * j, 16)])

        def swait(par):
            pltpu.make_async_copy(
                msg_buf[par], acc.at[didx_s[par]], semS[par]).wait()

        def sfire(par):
            for q in range(CHUNK // 16):
                didx_s[par][pl.ds(16 * q, 16)] = (
                    didx_c[par][pl.ds(16 * q, 16)])
            pltpu.async_copy(msg_buf[par], acc.at[didx_s[par]], semS[par],
                             add=True)

        for par in range(2):
            stage_idx(par, par)
            fire(par)
        for par in range(2):
            gwait(par)
            compute(par)
            sfire(par)
            stage_idx(par + 2, par)
            fire(par)

        def body(ci2, _):
            for par in range(2):
                ci = 2 * ci2 + par
                gwait(par)
                swait(par)
                compute(par)
                sfire(par)
                stage_idx(ci + 2, par)
                fire(par)
            return 0
        lax.fori_loop(1, chunks_per_tile // 2 - 1, body, 0)

        for par in range(2):
            gwait(par)
            swait(par)
            compute(par)
            sfire(par)
        for par in range(2):
            swait(par)
        plsc.subcore_barrier()

        for k in range(ROWS_PER_TILE // CHUNK):
            r = row0 + k * CHUNK
            pltpu.sync_copy(acc.at[pl.ds(r, CHUNK)], msg_buf[0])
            pltpu.sync_copy(msg_buf[0],
                            out_hbm.at[pl.ds(cid * NPAD + r, CHUNK)])

    return edge_kernel(t, src2d, dst2d).reshape(2, NPAD, w_row)


def kernel(x, edge_index, W1, a1_src, a1_dst, W2, a2_src, a2_dst):
    x = x.astype(jnp.float32)
    xp = jnp.pad(x, ((0, NPAD - N), (0, 0)))

    src = edge_index[0].astype(jnp.int32)
    dst = edge_index[1].astype(jnp.int32)
    pad_e = EPAD - E
    pad_idx = N + (jnp.arange(pad_e, dtype=jnp.int32) % (NPAD - N))
    src = jnp.concatenate([src, pad_idx])
    dst = jnp.concatenate([dst, pad_idx])
    src2d = src.reshape(EPAD // CHUNK, CHUNK)
    dst2d = dst.reshape(EPAD // CHUNK, CHUNK)

    eye1 = jnp.eye(H1, dtype=jnp.float32)
    a1s = (eye1[:, None, :] * a1_src[:, :, None]).reshape(H1 * F1, H1)
    a1d = (eye1[:, None, :] * a1_dst[:, :, None]).reshape(H1 * F1, H1)
    A1 = jnp.concatenate([a1s, a1d], axis=1)
    P = jnp.repeat(eye1, F1, axis=1)
    A2 = jnp.concatenate([a2_src.reshape(C, 1), a2_dst.reshape(C, 1)], axis=1)

    t1 = _tc_pre(xp, W1, A1, H1 * F1, H1)
    acc1 = _sc_edge_pass(t1, src2d, dst2d, H1 * F1, H1)
    t2 = _tc_mid(acc1[0], acc1[1], P, W2, A2)
    acc2 = _sc_edge_pass(t2, src2d, dst2d, C, 1)
    out = _tc_post(acc2[0], acc2[1])
    return out[:N]


# speedup vs baseline: 1.0025x; 1.0025x over previous
import functools

import jax
import jax.numpy as jnp
from jax import lax
from jax.experimental import pallas as pl
from jax.experimental.pallas import tpu as pltpu
from jax.experimental.pallas import tpu_sc as plsc

N = 10000
E = 320000
F_IN = 128
C = 16
H1 = 8
F1 = 8

NPAD = 10240
NTILES = 32
EPAD = 327680
EDGES_PER_TILE = EPAD // NTILES
CHUNK = 64
ROWS_PER_TILE = NPAD // 16
BLK = 512
GRID = NPAD // BLK
NEG = -1e30


def _tc_pre_body(x_ref, w_ref, a_ref, t_ref, *, f_out, h_heads):
    h = jnp.dot(x_ref[...], w_ref[...], preferred_element_type=jnp.float32)
    al = jnp.dot(h, a_ref[...], preferred_element_type=jnp.float32)
    n = h.shape[0]
    z = jnp.zeros((n, 16 - h_heads), jnp.float32)
    neg = jnp.full((n, 16 - h_heads), NEG, jnp.float32)
    ztail = jnp.zeros((n, 128 - f_out - 32), jnp.float32)
    t_ref[...] = jnp.concatenate(
        [h, al[:, :h_heads], z, al[:, h_heads:], neg, ztail], axis=1)


def _tc_pre(x, w, a, f_out, h_heads):
    f_in = x.shape[1]
    return pl.pallas_call(
        functools.partial(_tc_pre_body, f_out=f_out, h_heads=h_heads),
        grid=(GRID,),
        in_specs=[
            pl.BlockSpec((BLK, f_in), lambda i: (i, 0)),
            pl.BlockSpec((f_in, f_out), lambda i: (0, 0)),
            pl.BlockSpec((f_out, 2 * h_heads), lambda i: (0, 0)),
        ],
        out_specs=pl.BlockSpec((BLK, 128), lambda i: (i, 0)),
        out_shape=jax.ShapeDtypeStruct((NPAD, 128), jnp.float32),
    )(x, w, a)


def _tc_mid_body(a0_ref, a1_ref, p_ref, w2_ref, a2_ref, t_ref):
    a0 = a0_ref[...]
    a1 = a1_ref[...]
    num = a0[:, :64] + a1[:, :64]
    den = a0[:, 64:72] + a1[:, 64:72]
    den_e = jnp.dot(den, p_ref[...], preferred_element_type=jnp.float32)
    out1 = num / (den_e + 1e-16)
    g = jnp.where(out1 > 0, out1, jnp.exp(out1) - 1.0)
    h2 = jnp.dot(g, w2_ref[...], preferred_element_type=jnp.float32)
    al2 = jnp.dot(h2, a2_ref[...], preferred_element_type=jnp.float32)
    n = h2.shape[0]
    z = jnp.zeros((n, 15), jnp.float32)
    neg = jnp.full((n, 15), NEG, jnp.float32)
    ztail = jnp.zeros((n, 80), jnp.float32)
    t_ref[...] = jnp.concatenate(
        [h2, al2[:, 0:1], z, al2[:, 1:2], neg, ztail], axis=1)


def _tc_mid(acc0, acc1, p, w2, a2):
    return pl.pallas_call(
        _tc_mid_body,
        grid=(GRID,),
        in_specs=[
            pl.BlockSpec((BLK, 80), lambda i: (i, 0)),
            pl.BlockSpec((BLK, 80), lambda i: (i, 0)),
            pl.BlockSpec((8, 64), lambda i: (0, 0)),
            pl.BlockSpec((64, 16), lambda i: (0, 0)),
            pl.BlockSpec((16, 2), lambda i: (0, 0)),
        ],
        out_specs=pl.BlockSpec((BLK, 128), lambda i: (i, 0)),
        out_shape=jax.ShapeDtypeStruct((NPAD, 128), jnp.float32),
    )(acc0, acc1, p, w2, a2)


def _tc_post_body(a0_ref, a1_ref, out_ref):
    a0 = a0_ref[...]
    a1 = a1_ref[...]
    num = a0[:, :16] + a1[:, :16]
    den = a0[:, 16:17] + a1[:, 16:17]
    zv = num / (den + 1e-16)
    zm = jnp.max(zv, axis=1, keepdims=True)
    ez = jnp.exp(zv - zm)
    out_ref[...] = ez / jnp.sum(ez, axis=1, keepdims=True)


def _tc_post(acc0, acc1):
    return pl.pallas_call(
        _tc_post_body,
        grid=(GRID,),
        in_specs=[
            pl.BlockSpec((BLK, 32), lambda i: (i, 0)),
            pl.BlockSpec((BLK, 32), lambda i: (i, 0)),
        ],
        out_specs=pl.BlockSpec((BLK, 16), lambda i: (i, 0)),
        out_shape=jax.ShapeDtypeStruct((NPAD, 16), jnp.float32),
    )(acc0, acc1)


def _sc_edge_pass(t, src2d, dst2d, f_out, h_heads):
    w_row = f_out + 16
    nreg = f_out // 16
    shift = {8: 3, 16: 4}[f_out // h_heads]
    chunks_per_tile = EDGES_PER_TILE // CHUNK

    mesh = plsc.VectorSubcoreMesh(
        core_axis_name="c", subcore_axis_name="s", num_cores=2, num_subcores=16)

    @functools.partial(
        pl.kernel,
        out_type=jax.ShapeDtypeStruct((2 * NPAD, w_row), jnp.float32),
        mesh=mesh,
        compiler_params=pltpu.CompilerParams(use_tc_tiling_on_sc=False),
        scratch_types=[
            pltpu.VMEM((chunks_per_tile, CHUNK), jnp.int32),
            pltpu.VMEM((chunks_per_tile, CHUNK), jnp.int32),
            pltpu.VMEM((CHUNK, 128), jnp.float32),
            pltpu.VMEM((CHUNK, 128), jnp.float32),
            pltpu.VMEM((CHUNK, 128), jnp.float32),
            pltpu.VMEM((CHUNK, 128), jnp.float32),
            pltpu.VMEM((CHUNK, w_row), jnp.float32),
            pltpu.VMEM((CHUNK, w_row), jnp.float32),
            pltpu.VMEM((CHUNK,), jnp.int32),
            pltpu.VMEM((CHUNK,), jnp.int32),
            pltpu.VMEM((CHUNK,), jnp.int32),
            pltpu.VMEM((CHUNK,), jnp.int32),
            pltpu.VMEM((CHUNK,), jnp.int32),
            pltpu.VMEM((CHUNK,), jnp.int32),
            pltpu.VMEM_SHARED((NPAD, w_row), jnp.float32),
            pltpu.SemaphoreType.DMA,
            pltpu.SemaphoreType.DMA,
            pltpu.SemaphoreType.DMA,
            pltpu.SemaphoreType.DMA,
            pltpu.SemaphoreType.DMA,
            pltpu.SemaphoreType.DMA,
        ],
    )
    def edge_kernel(t_hbm, src_hbm, dst_hbm, out_hbm,
                    sidx, didx, abuf0, abuf1, bbuf0, bbuf1, msg0, msg1,
                    sidxc0, sidxc1, didxc0, didxc1, didxs0, didxs1,
                    acc, semA0, semA1, semB0, semB1, semS0, semS1):
        abuf = (abuf0, abuf1)
        bbuf = (bbuf0, bbuf1)
        msg_buf = (msg0, msg1)
        sidx_c = (sidxc0, sidxc1)
        didx_c = (didxc0, didxc1)
        didx_s = (didxs0, didxs1)
        semA = (semA0, semA1)
        semB = (semB0, semB1)
        semS = (semS0, semS1)
        cid = lax.axis_index("c")
        sid = lax.axis_index("s")
        wid = cid * 16 + sid

        def zrow(r, _):
            for j in range(w_row // 16):
                msg_buf[0][r, pl.ds(16 * j, 16)] = jnp.zeros((16,),
                                                             jnp.float32)
            return 0
        lax.fori_loop(0, CHUNK, zrow, 0)
        row0 = sid * ROWS_PER_TILE
        for k in range(ROWS_PER_TILE // CHUNK):
            pltpu.sync_copy(msg_buf[0],
                            acc.at[pl.ds(row0 + k * CHUNK, CHUNK)])

        pltpu.sync_copy(src_hbm.at[pl.ds(wid * chunks_per_tile,
                                         chunks_per_tile)], sidx)
        pltpu.sync_copy(dst_hbm.at[pl.ds(wid * chunks_per_tile,
                                         chunks_per_tile)], didx)
        plsc.subcore_barrier()

        def stage_idx(ci, par):
            for q in range(CHUNK // 16):
                sidx_c[par][pl.ds(16 * q, 16)] = sidx[ci, pl.ds(16 * q, 16)]
                didx_c[par][pl.ds(16 * q, 16)] = didx[ci, pl.ds(16 * q, 16)]

        def fire(par):
            pltpu.async_copy(t_hbm.at[sidx_c[par]], abuf[par], semA[par])
            pltpu.async_copy(t_hbm.at[didx_c[par]], bbuf[par], semB[par])

        def gwait(par):
            pltpu.make_async_copy(
                t_hbm.at[sidx_c[par]], abuf[par], semA[par]).wait()
            pltpu.make_async_copy(
                t_hbm.at[didx_c[par]], bbuf[par], semB[par]).wait()

        def compute(par):
            @plsc.parallel_loop(0, CHUNK, 1, unroll=4)
            def edge(b):
                asv = abuf[par][b, pl.ds(f_out, 16)]
                adv = bbuf[par][b, pl.ds(f_out + 16, 16)]
                e = asv + adv
                e = jnp.maximum(e, 0.2 * e)
                w = jnp.exp(e)
                msg_buf[par][b, pl.ds(f_out, 16)] = w
                lane = lax.iota(jnp.int32, 16)
                for j in range(nreg):
                    rep = lax.shift_right_logical(lane + 16 * j, shift)
                    wj = jnp.take_along_axis(w, rep, axis=0)
                    msg_buf[par][b, pl.ds(16 * j, 16)] = (
                        wj * abuf[par][b, pl.ds(16 * j, 16)])

        def swait(par):
            pltpu.make_async_copy(
                msg_buf[par], acc.at[didx_s[par]], semS[par]).wait()

        def sfire(par):
            for q in range(CHUNK // 16):
                didx_s[par][pl.ds(16 * q, 16)] = (
                    didx_c[par][pl.ds(16 * q, 16)])
            pltpu.async_copy(msg_buf[par], acc.at[didx_s[par]], semS[par],
                             add=True)

        for par in range(2):
            stage_idx(par, par)
            fire(par)
        for par in range(2):
            gwait(par)
            compute(par)
            sfire(par)
            stage_idx(par + 2, par)
            fire(par)

        def body(ci2, _):
            for par in range(2):
                ci = 2 * ci2 + par
                gwait(par)
                swait(par)
                compute(par)
                sfire(par)
                stage_idx(ci + 2, par)
                fire(par)
            return 0
        lax.fori_loop(1, chunks_per_tile // 2 - 1, body, 0)

        for par in range(2):
            gwait(par)
            swait(par)
            compute(par)
            sfire(par)
        for par in range(2):
            swait(par)
        plsc.subcore_barrier()

        for k in range(ROWS_PER_TILE // CHUNK):
            r = row0 + k * CHUNK
            pltpu.sync_copy(acc.at[pl.ds(r, CHUNK)], msg_buf[0])
            pltpu.sync_copy(msg_buf[0],
                            out_hbm.at[pl.ds(cid * NPAD + r, CHUNK)])

    return edge_kernel(t, src2d, dst2d).reshape(2, NPAD, w_row)


def kernel(x, edge_index, W1, a1_src, a1_dst, W2, a2_src, a2_dst):
    x = x.astype(jnp.float32)
    xp = jnp.pad(x, ((0, NPAD - N), (0, 0)))

    src = edge_index[0].astype(jnp.int32)
    dst = edge_index[1].astype(jnp.int32)
    pad_e = EPAD - E
    pad_idx = N + (jnp.arange(pad_e, dtype=jnp.int32) % (NPAD - N))
    src = jnp.concatenate([src, pad_idx])
    dst = jnp.concatenate([dst, pad_idx])
    src2d = src.reshape(EPAD // CHUNK, CHUNK)
    dst2d = dst.reshape(EPAD // CHUNK, CHUNK)

    eye1 = jnp.eye(H1, dtype=jnp.float32)
    a1s = (eye1[:, None, :] * a1_src[:, :, None]).reshape(H1 * F1, H1)
    a1d = (eye1[:, None, :] * a1_dst[:, :, None]).reshape(H1 * F1, H1)
    A1 = jnp.concatenate([a1s, a1d], axis=1)
    P = jnp.repeat(eye1, F1, axis=1)
    A2 = jnp.concatenate([a2_src.reshape(C, 1), a2_dst.reshape(C, 1)], axis=1)

    t1 = _tc_pre(xp, W1, A1, H1 * F1, H1)
    acc1 = _sc_edge_pass(t1, src2d, dst2d, H1 * F1, H1)
    t2 = _tc_mid(acc1[0], acc1[1], P, W2, A2)
    acc2 = _sc_edge_pass(t2, src2d, dst2d, C, 1)
    out = _tc_post(acc2[0], acc2[1])
    return out[:N]


# speedup vs baseline: 1.5390x; 1.5352x over previous
import functools

import jax
import jax.numpy as jnp
from jax import lax
from jax.experimental import pallas as pl
from jax.experimental.pallas import tpu as pltpu
from jax.experimental.pallas import tpu_sc as plsc

N = 10000
E = 320000
F_IN = 128
C = 16
H1 = 8
F1 = 8

NPAD = 10240
NTILES = 32
EPAD = 327680
EDGES_PER_TILE = EPAD // NTILES
CHUNK = 128
ROWS_PER_TILE = NPAD // 16
BLK = 512
GRID = NPAD // BLK
NEG = -1e30


def _tc_pre_body(x_ref, w_ref, a_ref, hs_ref, ad_ref, *, f_out, h_heads):
    h = jnp.dot(x_ref[...], w_ref[...], preferred_element_type=jnp.float32)
    al = jnp.dot(h, a_ref[...], preferred_element_type=jnp.float32)
    n = h.shape[0]
    z = jnp.zeros((n, 16 - h_heads), jnp.float32)
    neg = jnp.full((n, 16 - h_heads), NEG, jnp.float32)
    hs_ref[...] = jnp.concatenate([h, al[:, :h_heads], z], axis=1)
    ad_ref[...] = jnp.concatenate([al[:, h_heads:], neg], axis=1)


def _tc_pre(x, w, a, f_out, h_heads):
    f_in = x.shape[1]
    return pl.pallas_call(
        functools.partial(_tc_pre_body, f_out=f_out, h_heads=h_heads),
        grid=(GRID,),
        in_specs=[
            pl.BlockSpec((BLK, f_in), lambda i: (i, 0)),
            pl.BlockSpec((f_in, f_out), lambda i: (0, 0)),
            pl.BlockSpec((f_out, 2 * h_heads), lambda i: (0, 0)),
        ],
        out_specs=[
            pl.BlockSpec((BLK, f_out + 16), lambda i: (i, 0)),
            pl.BlockSpec((BLK, 16), lambda i: (i, 0)),
        ],
        out_shape=[
            jax.ShapeDtypeStruct((NPAD, f_out + 16), jnp.float32),
            jax.ShapeDtypeStruct((NPAD, 16), jnp.float32),
        ],
    )(x, w, a)


def _tc_mid_body(a0_ref, a1_ref, p_ref, w2_ref, a2_ref, hs_ref, ad_ref):
    a0 = a0_ref[...]
    a1 = a1_ref[...]
    num = a0[:, :64] + a1[:, :64]
    den = a0[:, 64:72] + a1[:, 64:72]
    den_e = jnp.dot(den, p_ref[...], preferred_element_type=jnp.float32)
    out1 = num / (den_e + 1e-16)
    g = jnp.where(out1 > 0, out1, jnp.exp(out1) - 1.0)
    h2 = jnp.dot(g, w2_ref[...], preferred_element_type=jnp.float32)
    al2 = jnp.dot(h2, a2_ref[...], preferred_element_type=jnp.float32)
    n = h2.shape[0]
    z = jnp.zeros((n, 15), jnp.float32)
    neg = jnp.full((n, 15), NEG, jnp.float32)
    hs_ref[...] = jnp.concatenate([h2, al2[:, 0:1], z], axis=1)
    ad_ref[...] = jnp.concatenate([al2[:, 1:2], neg], axis=1)


def _tc_mid(acc0, acc1, p, w2, a2):
    return pl.pallas_call(
        _tc_mid_body,
        grid=(GRID,),
        in_specs=[
            pl.BlockSpec((BLK, 80), lambda i: (i, 0)),
            pl.BlockSpec((BLK, 80), lambda i: (i, 0)),
            pl.BlockSpec((8, 64), lambda i: (0, 0)),
            pl.BlockSpec((64, 16), lambda i: (0, 0)),
            pl.BlockSpec((16, 2), lambda i: (0, 0)),
        ],
        out_specs=[
            pl.BlockSpec((BLK, 32), lambda i: (i, 0)),
            pl.BlockSpec((BLK, 16), lambda i: (i, 0)),
        ],
        out_shape=[
            jax.ShapeDtypeStruct((NPAD, 32), jnp.float32),
            jax.ShapeDtypeStruct((NPAD, 16), jnp.float32),
        ],
    )(acc0, acc1, p, w2, a2)


def _tc_post_body(a0_ref, a1_ref, out_ref):
    a0 = a0_ref[...]
    a1 = a1_ref[...]
    num = a0[:, :16] + a1[:, :16]
    den = a0[:, 16:17] + a1[:, 16:17]
    zv = num / (den + 1e-16)
    zm = jnp.max(zv, axis=1, keepdims=True)
    ez = jnp.exp(zv - zm)
    out_ref[...] = ez / jnp.sum(ez, axis=1, keepdims=True)


def _tc_post(acc0, acc1):
    return pl.pallas_call(
        _tc_post_body,
        grid=(GRID,),
        in_specs=[
            pl.BlockSpec((BLK, 32), lambda i: (i, 0)),
            pl.BlockSpec((BLK, 32), lambda i: (i, 0)),
        ],
        out_specs=pl.BlockSpec((BLK, 16), lambda i: (i, 0)),
        out_shape=jax.ShapeDtypeStruct((NPAD, 16), jnp.float32),
    )(acc0, acc1)


def _sc_edge_pass(hs, ad, src2d, dst2d, f_out, h_heads):
    w_row = f_out + 16
    nreg = f_out // 16
    shift = {8: 3, 16: 4}[f_out // h_heads]
    chunks_per_tile = EDGES_PER_TILE // CHUNK

    mesh = plsc.VectorSubcoreMesh(
        core_axis_name="c", subcore_axis_name="s", num_cores=2, num_subcores=16)

    @functools.partial(
        pl.kernel,
        out_type=jax.ShapeDtypeStruct((2 * NPAD, w_row), jnp.float32),
        mesh=mesh,
        compiler_params=pltpu.CompilerParams(use_tc_tiling_on_sc=False),
        scratch_types=[
            pltpu.VMEM((chunks_per_tile, CHUNK), jnp.int32),
            pltpu.VMEM((chunks_per_tile, CHUNK), jnp.int32),
            pltpu.VMEM((CHUNK, w_row), jnp.float32),
            pltpu.VMEM((CHUNK, w_row), jnp.float32),
            pltpu.VMEM((CHUNK, 16), jnp.float32),
            pltpu.VMEM((CHUNK, 16), jnp.float32),
            pltpu.VMEM((CHUNK, w_row), jnp.float32),
            pltpu.VMEM((CHUNK, w_row), jnp.float32),
            pltpu.VMEM((CHUNK,), jnp.int32),
            pltpu.VMEM((CHUNK,), jnp.int32),
            pltpu.VMEM((CHUNK,), jnp.int32),
            pltpu.VMEM((CHUNK,), jnp.int32),
            pltpu.VMEM((CHUNK,), jnp.int32),
            pltpu.VMEM((CHUNK,), jnp.int32),
            pltpu.VMEM_SHARED((NPAD, w_row), jnp.float32),
            pltpu.SemaphoreType.DMA,
            pltpu.SemaphoreType.DMA,
            pltpu.SemaphoreType.DMA,
            pltpu.SemaphoreType.DMA,
            pltpu.SemaphoreType.DMA,
            pltpu.SemaphoreType.DMA,
        ],
    )
    def edge_kernel(hs_hbm, ad_hbm, src_hbm, dst_hbm, out_hbm,
                    sidx, didx, abuf0, abuf1, bbuf0, bbuf1, msg0, msg1,
                    sidxc0, sidxc1, didxc0, didxc1, didxs0, didxs1,
                    acc, semA0, semA1, semB0, semB1, semS0, semS1):
        abuf = (abuf0, abuf1)
        bbuf = (bbuf0, bbuf1)
        msg_buf = (msg0, msg1)
        sidx_c = (sidxc0, sidxc1)
        didx_c = (didxc0, didxc1)
        didx_s = (didxs0, didxs1)
        semA = (semA0, semA1)
        semB = (semB0, semB1)
        semS = (semS0, semS1)
        cid = lax.axis_index("c")
        sid = lax.axis_index("s")
        wid = cid * 16 + sid

        def zrow(r, _):
            for j in range(w_row // 16):
                msg_buf[0][r, pl.ds(16 * j, 16)] = jnp.zeros((16,),
                                                             jnp.float32)
            return 0
        lax.fori_loop(0, CHUNK, zrow, 0)
        row0 = sid * ROWS_PER_TILE
        for k in range(ROWS_PER_TILE // CHUNK):
            pltpu.sync_copy(msg_buf[0],
                            acc.at[pl.ds(row0 + k * CHUNK, CHUNK)])

        pltpu.sync_copy(src_hbm.at[pl.ds(wid * chunks_per_tile,
                                         chunks_per_tile)], sidx)
        pltpu.sync_copy(dst_hbm.at[pl.ds(wid * chunks_per_tile,
                                         chunks_per_tile)], didx)
        plsc.subcore_barrier()

        def stage_idx(ci, par):
            for q in range(CHUNK // 16):
                sidx_c[par][pl.ds(16 * q, 16)] = sidx[ci, pl.ds(16 * q, 16)]
                didx_c[par][pl.ds(16 * q, 16)] = didx[ci, pl.ds(16 * q, 16)]

        def fire(par):
            pltpu.async_copy(hs_hbm.at[sidx_c[par]], abuf[par], semA[par])
            pltpu.async_copy(ad_hbm.at[didx_c[par]], bbuf[par], semB[par])

        def gwait(par):
            pltpu.make_async_copy(
                hs_hbm.at[sidx_c[par]], abuf[par], semA[par]).wait()
            pltpu.make_async_copy(
                ad_hbm.at[didx_c[par]], bbuf[par], semB[par]).wait()

        def compute(par):
            @plsc.parallel_loop(0, CHUNK, 1, unroll=4)
            def edge(b):
                asv = abuf[par][b, pl.ds(f_out, 16)]
                adv = bbuf[par][b, pl.ds(0, 16)]
                e = asv + adv
                e = jnp.maximum(e, 0.2 * e)
                w = jnp.exp(e)
                msg_buf[par][b, pl.ds(f_out, 16)] = w
                lane = lax.iota(jnp.int32, 16)
                for j in range(nreg):
                    rep = lax.shift_right_logical(lane + 16 * j, shift)
                    wj = jnp.take_along_axis(w, rep, axis=0)
                    msg_buf[par][b, pl.ds(16 * j, 16)] = (
                        wj * abuf[par][b, pl.ds(16 * j, 16)])

        def swait(par):
            pltpu.make_async_copy(
                msg_buf[par], acc.at[didx_s[par]], semS[par]).wait()

        def sfire(par):
            for q in range(CHUNK // 16):
                didx_s[par][pl.ds(16 * q, 16)] = (
                    didx_c[par][pl.ds(16 * q, 16)])
            pltpu.async_copy(msg_buf[par], acc.at[didx_s[par]], semS[par],
                             add=True)

        for par in range(2):
            stage_idx(par, par)
            fire(par)
        for par in range(2):
            gwait(par)
            compute(par)
            sfire(par)
            stage_idx(par + 2, par)
            fire(par)

        def body(ci2, _):
            for par in range(2):
                ci = 2 * ci2 + par
                gwait(par)
                swait(par)
                compute(par)
                sfire(par)
                stage_idx(ci + 2, par)
                fire(par)
            return 0
        lax.fori_loop(1, chunks_per_tile // 2 - 1, body, 0)

        for par in range(2):
            gwait(par)
            swait(par)
            compute(par)
            sfire(par)
        for par in range(2):
            swait(par)
        plsc.subcore_barrier()

        for k in range(ROWS_PER_TILE // CHUNK):
            r = row0 + k * CHUNK
            pltpu.sync_copy(acc.at[pl.ds(r, CHUNK)], msg_buf[0])
            pltpu.sync_copy(msg_buf[0],
                            out_hbm.at[pl.ds(cid * NPAD + r, CHUNK)])

    return edge_kernel(hs, ad, src2d, dst2d).reshape(2, NPAD, w_row)


def kernel(x, edge_index, W1, a1_src, a1_dst, W2, a2_src, a2_dst):
    x = x.astype(jnp.float32)
    xp = jnp.pad(x, ((0, NPAD - N), (0, 0)))

    src = edge_index[0].astype(jnp.int32)
    dst = edge_index[1].astype(jnp.int32)
    pad_e = EPAD - E
    pad_idx = N + (jnp.arange(pad_e, dtype=jnp.int32) % (NPAD - N))
    src = jnp.concatenate([src, pad_idx])
    dst = jnp.concatenate([dst, pad_idx])
    src2d = src.reshape(EPAD // CHUNK, CHUNK)
    dst2d = dst.reshape(EPAD // CHUNK, CHUNK)

    eye1 = jnp.eye(H1, dtype=jnp.float32)
    a1s = (eye1[:, None, :] * a1_src[:, :, None]).reshape(H1 * F1, H1)
    a1d = (eye1[:, None, :] * a1_dst[:, :, None]).reshape(H1 * F1, H1)
    A1 = jnp.concatenate([a1s, a1d], axis=1)
    P = jnp.repeat(eye1, F1, axis=1)
    A2 = jnp.concatenate([a2_src.reshape(C, 1), a2_dst.reshape(C, 1)], axis=1)

    hs1, ad1 = _tc_pre(xp, W1, A1, H1 * F1, H1)
    acc1 = _sc_edge_pass(hs1, ad1, src2d, dst2d, H1 * F1, H1)
    hs2, ad2 = _tc_mid(acc1[0], acc1[1], P, W2, A2)
    acc2 = _sc_edge_pass(hs2, ad2, src2d, dst2d, C, 1)
    out = _tc_post(acc2[0], acc2[1])
    return out[:N]
